# baseline (speedup 1.0000x reference)
import functools

import jax
import jax.numpy as jnp
from jax import lax
from jax.experimental import pallas as pl
from jax.experimental.pallas import tpu as pltpu
from jax.experimental.pallas import tpu_sc as plsc

N_ATOMS = 10000
MAX_NB = 32
N_BONDS = 320000
ATOM_FDIM = 128
BOND_FDIM = 144
HID = 300

L = 16
H = 384
NV = H // L
NC, NS = 2, 16
NW = NC * NS
NA = 10240
APW = NA // NW
BPW = N_BONDS // NW
CA = 2
CB = 80

_SC_MESH = dict(core_axis_name="c", subcore_axis_name="s")


def _mm_relu_body(x_ref, w_ref, o_ref):
    o_ref[...] = jnp.maximum(
        jnp.dot(x_ref[...], w_ref[...], preferred_element_type=jnp.float32), 0.0)


def mm_relu(x, w, bm):
    m, k = x.shape
    n = w.shape[1]
    return pl.pallas_call(
        _mm_relu_body,
        grid=(m // bm,),
        in_specs=[pl.BlockSpec((bm, k), lambda i: (i, 0)),
                  pl.BlockSpec((k, n), lambda i: (0, 0))],
        out_specs=pl.BlockSpec((bm, n), lambda i: (i, 0)),
        out_shape=jax.ShapeDtypeStruct((m, n), jnp.float32),
    )(x, w)


def _upd_body(ma_ref, agg_ref, man_ref):
    man_ref[...] = ma_ref[...] + agg_ref[...]


def upd_add(msg_atom, agg, bm):
    m = msg_atom.shape[0]
    return pl.pallas_call(
        _upd_body,
        grid=(m // bm,),
        in_specs=[pl.BlockSpec((bm, H), lambda i: (i, 0)),
                  pl.BlockSpec((bm, H), lambda i: (i, 0))],
        out_specs=pl.BlockSpec((bm, H), lambda i: (i, 0)),
        out_shape=jax.ShapeDtypeStruct((m, H), jnp.float32),
    )(msg_atom, agg)


def _comb_body(pre_ref, w_ref, ib_ref, o_ref):
    o_ref[...] = jnp.maximum(
        ib_ref[...]
        + jnp.dot(pre_ref[...], w_ref[...], preferred_element_type=jnp.float32),
        0.0)


def comb_mm(pre, w, input_bond, bm):
    m = pre.shape[0]
    return pl.pallas_call(
        _comb_body,
        grid=(m // bm,),
        in_specs=[pl.BlockSpec((bm, H), lambda i: (i, 0)),
                  pl.BlockSpec((H, H), lambda i: (0, 0)),
                  pl.BlockSpec((bm, H), lambda i: (i, 0))],
        out_specs=pl.BlockSpec((bm, H), lambda i: (i, 0)),
        out_shape=jax.ShapeDtypeStruct((m, H), jnp.float32),
    )(pre, w, input_bond)


def _final_body(xa_ref, xb_ref, xc_ref, wa_ref, wb_ref, wc_ref, o_ref):
    acc = jnp.dot(xa_ref[...], wa_ref[...], preferred_element_type=jnp.float32)
    acc += jnp.dot(xb_ref[...], wb_ref[...], preferred_element_type=jnp.float32)
    acc += jnp.dot(xc_ref[...], wc_ref[...], preferred_element_type=jnp.float32)
    o_ref[...] = jnp.maximum(acc, 0.0)


def final_mm(xa, xb, xc, wa, wb, wc, bm):
    m = xa.shape[0]
    xspec = pl.BlockSpec((bm, H), lambda i: (i, 0))
    wspec = pl.BlockSpec((H, H), lambda i: (0, 0))
    return pl.pallas_call(
        _final_body,
        grid=(m // bm,),
        in_specs=[xspec, xspec, xspec, wspec, wspec, wspec],
        out_specs=pl.BlockSpec((bm, H), lambda i: (i, 0)),
        out_shape=jax.ShapeDtypeStruct((m, H), jnp.float32),
    )(xa, xb, xc, wa, wb, wc)


def gather_reduce(msg_bond, a2b_flat):
    NCH = APW // CA
    NP = NCH // 2
    CROWS = CA * MAX_NB

    @functools.partial(
        pl.kernel,
        mesh=plsc.VectorSubcoreMesh(**_SC_MESH),
        out_type=jax.ShapeDtypeStruct((NA, H), jnp.float32),
        scratch_types=[
            pltpu.VMEM((CROWS,), jnp.int32),
            pltpu.VMEM((CROWS,), jnp.int32),
            pltpu.VMEM((CROWS, H), jnp.float32),
            pltpu.VMEM((CROWS, H), jnp.float32),
            pltpu.VMEM((8 * CA, H), jnp.float32),
            pltpu.VMEM((8 * CA, H), jnp.float32),
            pltpu.SemaphoreType.DMA,
            pltpu.SemaphoreType.DMA,
            pltpu.SemaphoreType.DMA,
            pltpu.SemaphoreType.DMA,
            pltpu.SemaphoreType.DMA,
            pltpu.SemaphoreType.DMA,
        ],
    )
    def k(msgb_hbm, a2b_hbm, agg_hbm, idx0, idx1, rows0, rows1, agg0, agg1,
          g0, g1, o0, o1, i0, i1):
        wid = lax.axis_index("s") * NC + lax.axis_index("c")
        idx = (idx0, idx1)
        rows = (rows0, rows1)
        agg = (agg0, agg1)
        gsem = (g0, g1)
        osem = (o0, o1)
        isem = (i0, i1)
        base_a = wid * APW

        def idesc(j, s):
            off = pl.multiple_of((base_a + j * CA) * MAX_NB, CROWS)
            return pltpu.make_async_copy(a2b_hbm.at[pl.ds(off, CROWS)],
                                         idx[s], isem[s])

        def gdesc(s):
            return pltpu.make_async_copy(msgb_hbm.at[idx[s]], rows[s],
                                         gsem[s])

        def compute(s, ro):
            rv = rows[s]

            def abody(a, c2):
                r0 = a * MAX_NB
                for v in range(NV):
                    sl = pl.ds(v * L, L)
                    x = rv[r0, sl]
                    sm = x
                    mx = x
                    for r in range(1, MAX_NB):
                        x = rv[r0 + r, sl]
                        sm = sm + x
                        mx = jnp.maximum(mx, x)
                    agg0[ro + a, sl] = sm * mx
                return c2

            lax.fori_loop(0, CA, abody, 0)

        def out_sync16(first_pair):
            off = pl.multiple_of(base_a + first_pair * 2 * CA, 8 * CA)
            pltpu.sync_copy(agg0, agg_hbm.at[pl.ds(off, 8 * CA)])

        idesc(0, 0).start()
        idesc(1, 1).start()
        idesc(0, 0).wait()
        gdesc(0).start()
        idesc(1, 1).wait()
        gdesc(1).start()

        def body(p, carry):
            j0 = 2 * p
            j1 = j0 + 1
            ro = (p % 4) * 2 * CA
            gdesc(0).wait()
            idesc(j0 + 2, 0).start()
            compute(0, ro)
            idesc(j0 + 2, 0).wait()
            gdesc(0).start()
            gdesc(1).wait()
            idesc(j1 + 2, 1).start()
            compute(1, ro + CA)
            idesc(j1 + 2, 1).wait()
            gdesc(1).start()

            @pl.when(p % 4 == 3)
            def _():
                out_sync16(p - 3)

            return carry

        lax.fori_loop(0, NP - 1, body, 0)
        p = NP - 1
        gdesc(0).wait()
        compute(0, (p % 4) * 2 * CA)
        gdesc(1).wait()
        compute(1, (p % 4) * 2 * CA + CA)
        out_sync16(p - 3)

    return k(msg_bond, a2b_flat)


def gather_sub(a2t, msg_bond, b2a, b2revb):
    NCH = BPW // CB
    NP = NCH // 2

    @functools.partial(
        pl.kernel,
        mesh=plsc.VectorSubcoreMesh(**_SC_MESH),
        out_type=jax.ShapeDtypeStruct((N_BONDS, H), jnp.float32),
        scratch_types=[
            pltpu.VMEM((CB,), jnp.int32),
            pltpu.VMEM((CB,), jnp.int32),
            pltpu.VMEM((CB,), jnp.int32),
            pltpu.VMEM((CB,), jnp.int32),
            pltpu.VMEM((CB, H), jnp.float32),
            pltpu.VMEM((CB, H), jnp.float32),
            pltpu.VMEM((CB, H), jnp.float32),
            pltpu.VMEM((CB, H), jnp.float32),
            pltpu.SemaphoreType.DMA,
            pltpu.SemaphoreType.DMA,
            pltpu.SemaphoreType.DMA,
            pltpu.SemaphoreType.DMA,
            pltpu.SemaphoreType.DMA,
            pltpu.SemaphoreType.DMA,
        ],
    )
    def k(a2_hbm, msgb_hbm, b2a_hbm, b2revb_hbm, pre_hbm,
          idxa0, idxa1, idxr0, idxr1, bufa0, bufa1, bufr0, bufr1,
          g0, g1, o0, o1, i0, i1):
        wid = lax.axis_index("s") * NC + lax.axis_index("c")
        idxa = (idxa0, idxa1)
        idxr = (idxr0, idxr1)
        iasem = (i0, i1)
        irsem = (o0, o1)
        del bufa1, bufr1, g1
        base_b = pl.multiple_of(wid * BPW, CB)

        def ia_desc(j, s):
            off = pl.multiple_of(base_b + j * CB, CB)
            return pltpu.make_async_copy(b2a_hbm.at[pl.ds(off, CB)],
                                         idxa[s], iasem[s])

        def ir_desc(j, s):
            off = pl.multiple_of(base_b + j * CB, CB)
            return pltpu.make_async_copy(b2revb_hbm.at[pl.ds(off, CB)],
                                         idxr[s], irsem[s])

        def istart(j, s):
            ia_desc(j, s).start()
            ir_desc(j, s).start()

        def iwait(j, s):
            ia_desc(j, s).wait()
            ir_desc(j, s).wait()

        def compute():
            def rbody(r, c2):
                for v in range(NV):
                    sl = pl.ds(v * L, L)
                    bufa0[r, sl] = bufa0[r, sl] - bufr0[r, sl]
                return c2

            lax.fori_loop(0, CB, rbody, 0)

        def chunk(j, s, prefetch):
            b0 = pl.multiple_of(base_b + j * CB, CB)
            iwait(j, s)
            pltpu.async_copy(a2_hbm.at[idxa[s]], bufa0, g0).wait()
            pltpu.async_copy(msgb_hbm.at[idxr[s]], bufr0, g0).wait()
            if prefetch:
                istart(j + 2, s)
            compute()
            pltpu.sync_copy(bufa0, pre_hbm.at[pl.ds(b0, CB)])

        istart(0, 0)
        istart(1, 1)

        def body(p, carry):
            j0 = 2 * p
            chunk(j0, 0, True)
            chunk(j0 + 1, 1, True)
            return carry

        lax.fori_loop(0, NP - 1, body, 0)
        chunk(NCH - 3, 0, True)
        chunk(NCH - 2, 1, False)
        chunk(NCH - 1, 0, False)

    return k(a2t, msg_bond, b2a, b2revb)


def _pad2(x, r, c):
    return jnp.pad(x, ((0, r - x.shape[0]), (0, c - x.shape[1])))


def kernel(f_atoms, f_bonds, a2b, b2a, b2revb,
           W_i_atom, W_i_bond, W_h_0, W_h_1, W_lr):
    f_atoms_p = _pad2(f_atoms, NA, ATOM_FDIM)
    wia = _pad2(W_i_atom, ATOM_FDIM, H)
    wib = _pad2(W_i_bond, BOND_FDIM, H)
    wh0 = _pad2(W_h_0, H, H)
    wh1 = _pad2(W_h_1, H, H)
    wl_a = _pad2(W_lr[0:HID], H, H)
    wl_m = _pad2(W_lr[HID:2 * HID], H, H)
    wl_i = _pad2(W_lr[2 * HID:3 * HID], H, H)

    a2b_flat = jnp.pad(a2b.astype(jnp.int32), ((0, NA - N_ATOMS), (0, 0)))
    a2b_flat = jnp.pad(a2b_flat.reshape(-1), (0, 2 * CA * MAX_NB))
    b2a32 = b2a.astype(jnp.int32)
    b2revb32 = b2revb.astype(jnp.int32)

    input_atom = mm_relu(f_atoms_p, wia, bm=1024)
    input_bond = mm_relu(f_bonds, wib, bm=2000)

    msg_atom = input_atom
    msg_bond = input_bond
    for wh in (wh0, wh1):
        agg = gather_reduce(msg_bond, a2b_flat)
        msg_atom = upd_add(msg_atom, agg, bm=1024)
        pre = gather_sub(msg_atom, msg_bond, b2a32, b2revb32)
        msg_bond = comb_mm(pre, wh, input_bond, bm=2000)

    agg2 = gather_reduce(msg_bond, a2b_flat)
    out = final_mm(agg2, msg_atom, input_atom, wl_a, wl_m, wl_i, bm=1024)
    return out[1:N_ATOMS, 0:HID]


# speedup vs baseline: 1.1612x; 1.1612x over previous
import functools

import jax
import jax.numpy as jnp
from jax import lax
from jax.experimental import pallas as pl
from jax.experimental.pallas import tpu as pltpu
from jax.experimental.pallas import tpu_sc as plsc

N_ATOMS = 10000
MAX_NB = 32
N_BONDS = 320000
ATOM_FDIM = 128
BOND_FDIM = 144
HID = 300

L = 16
H = 384
NV = H // L
NVU = 19
NC, NS = 2, 16
NW = NC * NS
NA = 10240
APW = NA // NW
BPW = N_BONDS // NW
CA = 2
CB = 80

_SC_MESH = dict(core_axis_name="c", subcore_axis_name="s")


def _bdot(x, w):
    return jnp.dot(x.astype(jnp.bfloat16), w.astype(jnp.bfloat16),
                   preferred_element_type=jnp.float32)


def _mm_relu_body(x_ref, w_ref, o_ref):
    o_ref[...] = jnp.maximum(_bdot(x_ref[...], w_ref[...]), 0.0)


def mm_relu(x, w, bm):
    m, k = x.shape
    n = w.shape[1]
    return pl.pallas_call(
        _mm_relu_body,
        grid=(m // bm,),
        in_specs=[pl.BlockSpec((bm, k), lambda i: (i, 0)),
                  pl.BlockSpec((k, n), lambda i: (0, 0))],
        out_specs=pl.BlockSpec((bm, n), lambda i: (i, 0)),
        out_shape=jax.ShapeDtypeStruct((m, n), jnp.float32),
    )(x, w)


def _upd_body(ma_ref, agg_ref, man_ref):
    man_ref[...] = ma_ref[...] + agg_ref[...]


def upd_add(msg_atom, agg, bm):
    m = msg_atom.shape[0]
    return pl.pallas_call(
        _upd_body,
        grid=(m // bm,),
        in_specs=[pl.BlockSpec((bm, H), lambda i: (i, 0)),
                  pl.BlockSpec((bm, H), lambda i: (i, 0))],
        out_specs=pl.BlockSpec((bm, H), lambda i: (i, 0)),
        out_shape=jax.ShapeDtypeStruct((m, H), jnp.float32),
    )(msg_atom, agg)


def _comb_body(pre_ref, w_ref, ib_ref, o_ref):
    o_ref[...] = jnp.maximum(ib_ref[...] + _bdot(pre_ref[...], w_ref[...]),
                             0.0)


def comb_mm(pre, w, input_bond, bm):
    m = pre.shape[0]
    return pl.pallas_call(
        _comb_body,
        grid=(m // bm,),
        in_specs=[pl.BlockSpec((bm, H), lambda i: (i, 0)),
                  pl.BlockSpec((H, H), lambda i: (0, 0)),
                  pl.BlockSpec((bm, H), lambda i: (i, 0))],
        out_specs=pl.BlockSpec((bm, H), lambda i: (i, 0)),
        out_shape=jax.ShapeDtypeStruct((m, H), jnp.float32),
    )(pre, w, input_bond)


def _final_body(xa_ref, xb_ref, xc_ref, wa_ref, wb_ref, wc_ref, o_ref):
    acc = _bdot(xa_ref[...], wa_ref[...])
    acc += _bdot(xb_ref[...], wb_ref[...])
    acc += _bdot(xc_ref[...], wc_ref[...])
    o_ref[...] = jnp.maximum(acc, 0.0)


def final_mm(xa, xb, xc, wa, wb, wc, bm):
    m = xa.shape[0]
    xspec = pl.BlockSpec((bm, H), lambda i: (i, 0))
    wspec = pl.BlockSpec((H, H), lambda i: (0, 0))
    return pl.pallas_call(
        _final_body,
        grid=(m // bm,),
        in_specs=[xspec, xspec, xspec, wspec, wspec, wspec],
        out_specs=pl.BlockSpec((bm, H), lambda i: (i, 0)),
        out_shape=jax.ShapeDtypeStruct((m, H), jnp.float32),
    )(xa, xb, xc, wa, wb, wc)


def gather_reduce(msg_bond, a2b_flat):
    NCH = APW // CA
    NP = NCH // 2
    CROWS = CA * MAX_NB

    @functools.partial(
        pl.kernel,
        mesh=plsc.VectorSubcoreMesh(**_SC_MESH),
        out_type=jax.ShapeDtypeStruct((NA, H), jnp.float32),
        scratch_types=[
            pltpu.VMEM((CROWS,), jnp.int32),
            pltpu.VMEM((CROWS,), jnp.int32),
            pltpu.VMEM((CROWS, H), jnp.float32),
            pltpu.VMEM((CROWS, H), jnp.float32),
            pltpu.VMEM((8 * CA, H), jnp.float32),
            pltpu.VMEM((8 * CA, H), jnp.float32),
            pltpu.SemaphoreType.DMA,
            pltpu.SemaphoreType.DMA,
            pltpu.SemaphoreType.DMA,
            pltpu.SemaphoreType.DMA,
            pltpu.SemaphoreType.DMA,
            pltpu.SemaphoreType.DMA,
        ],
    )
    def k(msgb_hbm, a2b_hbm, agg_hbm, idx0, idx1, rows0, rows1, agg0, agg1,
          g0, g1, o0, o1, i0, i1):
        wid = lax.axis_index("s") * NC + lax.axis_index("c")
        idx = (idx0, idx1)
        rows = (rows0, rows1)
        agg = (agg0, agg1)
        gsem = (g0, g1)
        osem = (o0, o1)
        isem = (i0, i1)
        base_a = wid * APW

        def idesc(j, s):
            off = pl.multiple_of((base_a + j * CA) * MAX_NB, CROWS)
            return pltpu.make_async_copy(a2b_hbm.at[pl.ds(off, CROWS)],
                                         idx[s], isem[s])

        def gdesc(s):
            return pltpu.make_async_copy(msgb_hbm.at[idx[s]], rows[s],
                                         gsem[s])

        def compute(s, ro):
            rv = rows[s]

            def abody(a, c2):
                r0 = a * MAX_NB
                for v in range(NVU):
                    sl = pl.ds(v * L, L)
                    x = rv[r0, sl]
                    sm = x
                    mx = x
                    for r in range(1, MAX_NB):
                        x = rv[r0 + r, sl]
                        sm = sm + x
                        mx = jnp.maximum(mx, x)
                    agg0[ro + a, sl] = sm * mx
                return c2

            lax.fori_loop(0, CA, abody, 0)

        def out_sync16(first_pair):
            off = pl.multiple_of(base_a + first_pair * 2 * CA, 8 * CA)
            pltpu.sync_copy(agg0, agg_hbm.at[pl.ds(off, 8 * CA)])

        zero = jnp.zeros((L,), jnp.float32)
        for rr in range(8 * CA):
            for v in range(NVU, NV):
                agg0[rr, pl.ds(v * L, L)] = zero

        idesc(0, 0).start()
        idesc(1, 1).start()
        idesc(0, 0).wait()
        gdesc(0).start()
        idesc(1, 1).wait()
        gdesc(1).start()

        def body(p, carry):
            j0 = 2 * p
            j1 = j0 + 1
            ro = (p % 4) * 2 * CA
            gdesc(0).wait()
            idesc(j0 + 2, 0).start()
            compute(0, ro)
            idesc(j0 + 2, 0).wait()
            gdesc(0).start()
            gdesc(1).wait()
            idesc(j1 + 2, 1).start()
            compute(1, ro + CA)
            idesc(j1 + 2, 1).wait()
            gdesc(1).start()

            @pl.when(p % 4 == 3)
            def _():
                out_sync16(p - 3)

            return carry

        lax.fori_loop(0, NP - 1, body, 0)
        p = NP - 1
        gdesc(0).wait()
        compute(0, (p % 4) * 2 * CA)
        gdesc(1).wait()
        compute(1, (p % 4) * 2 * CA + CA)
        out_sync16(p - 3)

    return k(msg_bond, a2b_flat)


def gather_sub(a2t, msg_bond, b2a, b2revb):
    NCH = BPW // CB
    NP = NCH // 2

    @functools.partial(
        pl.kernel,
        mesh=plsc.VectorSubcoreMesh(**_SC_MESH),
        out_type=jax.ShapeDtypeStruct((N_BONDS, H), jnp.float32),
        scratch_types=[
            pltpu.VMEM((CB,), jnp.int32),
            pltpu.VMEM((CB,), jnp.int32),
            pltpu.VMEM((CB,), jnp.int32),
            pltpu.VMEM((CB,), jnp.int32),
            pltpu.VMEM((CB, H), jnp.float32),
            pltpu.VMEM((CB, H), jnp.float32),
            pltpu.VMEM((CB, H), jnp.float32),
            pltpu.VMEM((CB, H), jnp.float32),
            pltpu.SemaphoreType.DMA,
            pltpu.SemaphoreType.DMA,
            pltpu.SemaphoreType.DMA,
            pltpu.SemaphoreType.DMA,
            pltpu.SemaphoreType.DMA,
            pltpu.SemaphoreType.DMA,
        ],
    )
    def k(a2_hbm, msgb_hbm, b2a_hbm, b2revb_hbm, pre_hbm,
          idxa0, idxa1, idxr0, idxr1, bufa0, bufa1, bufr0, bufr1,
          g0, g1, o0, o1, i0, i1):
        wid = lax.axis_index("s") * NC + lax.axis_index("c")
        idxa = (idxa0, idxa1)
        idxr = (idxr0, idxr1)
        iasem = (i0, i1)
        irsem = (o0, o1)
        del bufa1, bufr1, g1
        base_b = pl.multiple_of(wid * BPW, CB)

        def ia_desc(j, s):
            off = pl.multiple_of(base_b + j * CB, CB)
            return pltpu.make_async_copy(b2a_hbm.at[pl.ds(off, CB)],
                                         idxa[s], iasem[s])

        def ir_desc(j, s):
            off = pl.multiple_of(base_b + j * CB, CB)
            return pltpu.make_async_copy(b2revb_hbm.at[pl.ds(off, CB)],
                                         idxr[s], irsem[s])

        def istart(j, s):
            ia_desc(j, s).start()
            ir_desc(j, s).start()

        def iwait(j, s):
            ia_desc(j, s).wait()
            ir_desc(j, s).wait()

        def compute():
            def rbody(r, c2):
                for v in range(NVU):
                    sl = pl.ds(v * L, L)
                    bufa0[r, sl] = bufa0[r, sl] - bufr0[r, sl]
                return c2

            lax.fori_loop(0, CB, rbody, 0)

        def chunk(j, s, prefetch):
            b0 = pl.multiple_of(base_b + j * CB, CB)
            iwait(j, s)
            pltpu.async_copy(a2_hbm.at[idxa[s]], bufa0, g0).wait()
            pltpu.async_copy(msgb_hbm.at[idxr[s]], bufr0, g0).wait()
            if prefetch:
                istart(j + 2, s)
            compute()
            pltpu.sync_copy(bufa0, pre_hbm.at[pl.ds(b0, CB)])

        istart(0, 0)
        istart(1, 1)

        def body(p, carry):
            j0 = 2 * p
            chunk(j0, 0, True)
            chunk(j0 + 1, 1, True)
            return carry

        lax.fori_loop(0, NP - 1, body, 0)
        chunk(NCH - 3, 0, True)
        chunk(NCH - 2, 1, False)
        chunk(NCH - 1, 0, False)

    return k(a2t, msg_bond, b2a, b2revb)


def _pad2(x, r, c):
    return jnp.pad(x, ((0, r - x.shape[0]), (0, c - x.shape[1])))


def kernel(f_atoms, f_bonds, a2b, b2a, b2revb,
           W_i_atom, W_i_bond, W_h_0, W_h_1, W_lr):
    f_atoms_p = _pad2(f_atoms, NA, ATOM_FDIM)
    wia = _pad2(W_i_atom, ATOM_FDIM, H)
    wib = _pad2(W_i_bond, BOND_FDIM, H)
    wh0 = _pad2(W_h_0, H, H)
    wh1 = _pad2(W_h_1, H, H)
    wl_a = _pad2(W_lr[0:HID], H, H)
    wl_m = _pad2(W_lr[HID:2 * HID], H, H)
    wl_i = _pad2(W_lr[2 * HID:3 * HID], H, H)

    a2b_flat = jnp.pad(a2b.astype(jnp.int32), ((0, NA - N_ATOMS), (0, 0)))
    a2b_flat = jnp.pad(a2b_flat.reshape(-1), (0, 2 * CA * MAX_NB))
    b2a32 = b2a.astype(jnp.int32)
    b2revb32 = b2revb.astype(jnp.int32)

    input_atom = mm_relu(f_atoms_p, wia, bm=1024)
    input_bond = mm_relu(f_bonds, wib, bm=2000)

    msg_atom = input_atom
    msg_bond = input_bond
    for wh in (wh0, wh1):
        agg = gather_reduce(msg_bond, a2b_flat)
        msg_atom = upd_add(msg_atom, agg, bm=1024)
        pre = gather_sub(msg_atom, msg_bond, b2a32, b2revb32)
        msg_bond = comb_mm(pre, wh, input_bond, bm=2000)

    agg2 = gather_reduce(msg_bond, a2b_flat)
    out = final_mm(agg2, msg_atom, input_atom, wl_a, wl_m, wl_i, bm=1024)
    return out[1:N_ATOMS, 0:HID]
